# baseline (speedup 1.0000x reference)
import functools

import jax
import jax.numpy as jnp
from jax import lax
from jax.experimental import pallas as pl
from jax.experimental.pallas import tpu as pltpu
from jax.experimental.pallas import tpu_sc as plsc

_NC = 2
_NS = 16
_NW = _NC * _NS
_R = 8
_SLOTS = 3


def _sc_body(E, S, B, T, tok_hbm, pos_hbm, out_hbm, *scr):
    nt = _SLOTS * B
    toks = scr[0:nt]
    poss = scr[nt:nt + _SLOTS]
    isems = scr[nt + _SLOTS:2 * nt + _SLOTS]
    osems = scr[2 * nt + _SLOTS:3 * nt + _SLOTS]
    psems = scr[3 * nt + _SLOTS:3 * nt + 2 * _SLOTS]

    w = lax.axis_index("s") * _NC + lax.axis_index("c")
    s0 = w * (S // _NW)

    in_d, out_d, pos_d = {}, {}, {}
    unwaited_out = set()

    def start_in(t):
        sl = (t % _SLOTS) * B
        rows = pl.ds(s0 + t * _R, _R)
        pos_d[t] = pltpu.async_copy(
            pos_hbm.at[rows, :], poss[t % _SLOTS], psems[t % _SLOTS])
        for b in range(B):
            in_d[(t, b)] = pltpu.async_copy(
                tok_hbm.at[b, rows, :], toks[sl + b], isems[sl + b])

    def wait_out(t):
        if t in unwaited_out:
            for b in range(B):
                out_d[(t, b)].wait()
            unwaited_out.discard(t)

    start_in(0)
    if T > 1:
        start_in(1)

    for t in range(T):
        sl = (t % _SLOTS) * B
        rows = pl.ds(s0 + t * _R, _R)
        pos_d[t].wait()
        for b in range(B):
            in_d[(t, b)].wait()
        pos_v = poss[t % _SLOTS]
        bufs = toks[sl:sl + B]

        @plsc.parallel_loop(0, _R * E, step=16, unroll=8)
        def _(i):
            r = i // E
            c = i % E
            p = pos_v[r, pl.ds(c, 16)]
            for bv in bufs:
                bv[r, pl.ds(c, 16)] = bv[r, pl.ds(c, 16)] + p

        for b in range(B):
            out_d[(t, b)] = pltpu.async_copy(
                toks[sl + b], out_hbm.at[b, rows, :], osems[sl + b])
        unwaited_out.add(t)

        if t + 2 < T:
            wait_out(t - 1)
            start_in(t + 2)

    for t in range(T):
        wait_out(t)


def kernel(token_embedding, pos_table):
    B, S, E = token_embedding.shape
    T = S // _NW // _R
    mesh = plsc.VectorSubcoreMesh(core_axis_name="c", subcore_axis_name="s")
    nt = _SLOTS * B
    scratch = (
        [pltpu.VMEM((_R, E), jnp.float32)] * (nt + _SLOTS)
        + [pltpu.SemaphoreType.DMA] * (2 * nt + _SLOTS)
    )
    k = pl.kernel(
        functools.partial(_sc_body, E, S, B, T),
        out_type=jax.ShapeDtypeStruct((B, S, E), token_embedding.dtype),
        mesh=mesh,
        scratch_types=scratch,
        compiler_params=pltpu.CompilerParams(use_tc_tiling_on_sc=True),
    )
    return k(token_embedding, pos_table[:S])


# speedup vs baseline: 1.0423x; 1.0423x over previous
import functools

import jax
import jax.numpy as jnp
from jax import lax
from jax.experimental import pallas as pl
from jax.experimental.pallas import tpu as pltpu
from jax.experimental.pallas import tpu_sc as plsc

_NC = 2
_NS = 16
_NW = _NC * _NS
_R = 16
_NBUF = 5


def _sc_body(E, S, B, T, tok_hbm, pos_hbm, out_hbm, *scr):
    toks = scr[0:_NBUF]
    poss = scr[_NBUF:_NBUF + 2]
    isems = scr[_NBUF + 2:2 * _NBUF + 2]
    osems = scr[2 * _NBUF + 2:3 * _NBUF + 2]
    psems = scr[3 * _NBUF + 2:3 * _NBUF + 4]

    w = lax.axis_index("s") * _NC + lax.axis_index("c")
    s0 = w * (S // _NW)
    N = T * B
    PF = _NBUF - 2

    in_d, out_d, pos_d = {}, {}, {}

    def rows_of(t):
        return pl.ds(s0 + t * _R, _R)

    def start_in(u):
        t, b = divmod(u, B)
        in_d[u] = pltpu.async_copy(
            tok_hbm.at[b, rows_of(t), :], toks[u % _NBUF], isems[u % _NBUF])

    def start_pos(t):
        pos_d[t] = pltpu.async_copy(
            pos_hbm.at[rows_of(t), :], poss[t % 2], psems[t % 2])

    start_pos(0)
    if T > 1:
        start_pos(1)
    for u in range(min(PF, N)):
        start_in(u)

    for u in range(N):
        t, b = divmod(u, B)
        if u + PF < N:
            if u + PF - _NBUF >= 0:
                out_d[u + PF - _NBUF].wait()
            start_in(u + PF)
        if b == 0:
            pos_d[t].wait()
        in_d[u].wait()
        tok_v, pos_v = toks[u % _NBUF], poss[t % 2]

        @plsc.parallel_loop(0, _R * E, step=16, unroll=8)
        def _(i):
            r = i // E
            c = i % E
            tok_v[r, pl.ds(c, 16)] = (
                tok_v[r, pl.ds(c, 16)] + pos_v[r, pl.ds(c, 16)])

        out_d[u] = pltpu.async_copy(
            toks[u % _NBUF], out_hbm.at[b, rows_of(t), :], osems[u % _NBUF])
        if b == B - 1 and t + 2 < T:
            start_pos(t + 2)

    for u in range(max(0, N - _NBUF), N):
        out_d[u].wait()


def kernel(token_embedding, pos_table):
    B, S, E = token_embedding.shape
    T = S // _NW // _R
    mesh = plsc.VectorSubcoreMesh(core_axis_name="c", subcore_axis_name="s")
    scratch = (
        [pltpu.VMEM((_R, E), jnp.float32)] * (_NBUF + 2)
        + [pltpu.SemaphoreType.DMA] * (2 * _NBUF + 2)
    )
    k = pl.kernel(
        functools.partial(_sc_body, E, S, B, T),
        out_type=jax.ShapeDtypeStruct((B, S, E), token_embedding.dtype),
        mesh=mesh,
        scratch_types=scratch,
        compiler_params=pltpu.CompilerParams(use_tc_tiling_on_sc=True),
    )
    return k(token_embedding, pos_table[:S])
